# Initial kernel scaffold; baseline (speedup 1.0000x reference)
import functools

import jax
import jax.numpy as jnp
from jax import lax
from jax.experimental import pallas as pl
from jax.experimental.pallas import tpu as pltpu
from jax.experimental.pallas import tpu_sc as plsc

N, E, D, H = 10000, 320000, 128, 64
NC, NS = 2, 16
NPAD = NS * 640
ROWS_T = N // NS
EP_T = E // (NC * NS)
EP_S = E // NS
K = 128
EPS = 1e-5

_mesh = plsc.VectorSubcoreMesh(
    core_axis_name="c", subcore_axis_name="s", num_cores=NC, num_subcores=NS
)


@functools.partial(
    pl.kernel,
    out_type=jax.ShapeDtypeStruct((NC, NS, 640), jnp.float32),
    mesh=_mesh,
    scratch_types=[
        pltpu.VMEM_SHARED((NPAD,), jnp.float32),
        pltpu.VMEM((640,), jnp.float32),
        pltpu.VMEM((K,), jnp.int32),
        pltpu.VMEM((K,), jnp.float32),
        pltpu.VMEM((16,), jnp.int32),
        pltpu.VMEM((16,), jnp.float32),
    ],
)
def _deg_kernel(col_hbm, out_hbm, deg_sh, stage_v, idx_v, ones_v, idx_t, ones_t):
    c = lax.axis_index("c")
    s = lax.axis_index("s")
    ones16 = jnp.ones((16,), jnp.float32)
    for j in range(K // 16):
        ones_v[pl.ds(j * 16, 16)] = ones16
    ones_t[...] = ones16
    initv = jnp.where(c == 0, 1.0, 0.0).astype(jnp.float32) * ones16
    for j in range(640 // 16):
        stage_v[pl.ds(j * 16, 16)] = initv
    pltpu.sync_copy(stage_v, deg_sh.at[pl.ds(s * 640, 640)])
    plsc.subcore_barrier()

    ebase = (c * NS + s) * EP_T

    def chunk(g, carry):
        off = ebase + g * K
        pltpu.sync_copy(col_hbm.at[pl.ds(off, K)], idx_v)
        pltpu.sync_copy(ones_v, deg_sh.at[idx_v], add=True)
        return carry

    lax.fori_loop(0, EP_T // K, chunk, 0)
    off = ebase + (EP_T // K) * K
    pltpu.sync_copy(col_hbm.at[pl.ds(off, 16)], idx_t)
    pltpu.sync_copy(ones_t, deg_sh.at[idx_t], add=True)

    plsc.subcore_barrier()
    pltpu.sync_copy(deg_sh.at[pl.ds(s * 640, 640)], stage_v)
    pltpu.sync_copy(stage_v, out_hbm.at[c, s])


@functools.partial(
    pl.kernel,
    out_type=jax.ShapeDtypeStruct((NC * N, H), jnp.float32),
    mesh=_mesh,
    scratch_types=[
        pltpu.VMEM_SHARED((N, H), jnp.float32),
        pltpu.VMEM((ROWS_T, H), jnp.float32),
        pltpu.VMEM((K,), jnp.int32),
        pltpu.VMEM((K,), jnp.int32),
        pltpu.VMEM((K, H), jnp.float32),
        pltpu.VMEM((32,), jnp.int32),
        pltpu.VMEM((32,), jnp.int32),
        pltpu.VMEM((32, H), jnp.float32),
        pltpu.SemaphoreType.DMA,
    ],
)
def _msg_kernel(y_hbm, rowcat_hbm, col_hbm, out_hbm, acc_sh, stage_v,
                ridx_v, cidx_v, gath_v, ridx_t, cidx_t, gath_t, sem):
    c = lax.axis_index("c")
    s = lax.axis_index("s")
    node0 = s * ROWS_T
    pltpu.sync_copy(y_hbm.at[pl.ds(c * N + node0, ROWS_T)], stage_v)
    pltpu.sync_copy(stage_v, acc_sh.at[pl.ds(node0, ROWS_T)])
    plsc.subcore_barrier()

    ebase = s * EP_S

    def chunk(g, carry):
        off = ebase + g * K
        pltpu.sync_copy(rowcat_hbm.at[c, pl.ds(off, K)], ridx_v)
        pltpu.sync_copy(col_hbm.at[pl.ds(off, K)], cidx_v)
        pltpu.async_copy(y_hbm.at[ridx_v], gath_v, sem).wait()
        pltpu.sync_copy(gath_v, acc_sh.at[cidx_v], add=True)
        return carry

    lax.fori_loop(0, EP_S // K, chunk, 0)
    off = ebase + (EP_S // K) * K
    pltpu.sync_copy(rowcat_hbm.at[c, pl.ds(off, 32)], ridx_t)
    pltpu.sync_copy(col_hbm.at[pl.ds(off, 32)], cidx_t)
    pltpu.async_copy(y_hbm.at[ridx_t], gath_t, sem).wait()
    pltpu.sync_copy(gath_t, acc_sh.at[cidx_t], add=True)

    plsc.subcore_barrier()
    pltpu.sync_copy(acc_sh.at[pl.ds(node0, ROWS_T)], stage_v)
    pltpu.sync_copy(stage_v, out_hbm.at[pl.ds(c * N + node0, ROWS_T)])


def _dis_from_parts(degp):
    deg = degp.reshape(NC, NPAD)[:, :N].sum(axis=0)
    return lax.rsqrt(deg)[:, None]


def _tc1_body(x_ref, w_ref, degp_ref, y_ref):
    dis = _dis_from_parts(degp_ref[...])
    xw = jnp.dot(x_ref[...], w_ref[...], preferred_element_type=jnp.float32)
    y = dis * xw
    y_ref[:N] = y[:, :H]
    y_ref[N:] = y[:, H:]


_tc1 = pl.pallas_call(
    _tc1_body, out_shape=jax.ShapeDtypeStruct((NC * N, H), jnp.float32)
)


def _norm_act(acc, dis, b, g, be, a):
    h = dis * acc + b
    mean = jnp.mean(h, axis=0)
    var = jnp.mean((h - mean) ** 2, axis=0)
    hn = g * (h - mean) / jnp.sqrt(var + EPS) + be
    return jnp.where(hn >= 0, hn, a * hn)


def _tc_mid_body(acc_ref, degp_ref, b_ref, g_ref, be_ref, a_ref, w_ref, y_ref):
    dis = _dis_from_parts(degp_ref[...])
    h = jnp.concatenate([acc_ref[:N], acc_ref[N:]], axis=1)
    hp = _norm_act(h, dis, b_ref[...], g_ref[...], be_ref[...], a_ref[...])
    xw = jnp.dot(hp, w_ref[...], preferred_element_type=jnp.float32)
    y = dis * xw
    y_ref[:N] = y[:, :H]
    y_ref[N:] = y[:, H:]


_tc_mid = pl.pallas_call(
    _tc_mid_body, out_shape=jax.ShapeDtypeStruct((NC * N, H), jnp.float32)
)


def _tc_out_body(acc_ref, degp_ref, b_ref, g_ref, be_ref, a_ref, o_ref):
    dis = _dis_from_parts(degp_ref[...])
    h = jnp.concatenate([acc_ref[:N], acc_ref[N:]], axis=1)
    o_ref[...] = _norm_act(h, dis, b_ref[...], g_ref[...], be_ref[...], a_ref[...])


_tc_out = pl.pallas_call(
    _tc_out_body, out_shape=jax.ShapeDtypeStruct((N, D), jnp.float32)
)


def kernel(x, edge_index, W1, b1, gamma1, beta1, a1, W2, b2, gamma2, beta2, a2):
    row = edge_index[0]
    col = edge_index[1]
    rowcat = jnp.stack([row, row + N])
    degp = _deg_kernel(col)
    b1r, g1r, be1r = b1.reshape(1, -1), gamma1.reshape(1, -1), beta1.reshape(1, -1)
    b2r, g2r, be2r = b2.reshape(1, -1), gamma2.reshape(1, -1), beta2.reshape(1, -1)
    a1r, a2r = a1.reshape(1, 1), a2.reshape(1, 1)

    y1 = _tc1(x, W1, degp)
    acc1 = _msg_kernel(y1, rowcat, col)
    y2 = _tc_mid(acc1, degp, b1r, g1r, be1r, a1r, W2)
    acc2 = _msg_kernel(y2, rowcat, col)
    return _tc_out(acc2, degp, b2r, g2r, be2r, a2r)


# speedup vs baseline: 16.4365x; 16.4365x over previous
import functools

import jax
import jax.numpy as jnp
from jax import lax
from jax.experimental import pallas as pl
from jax.experimental.pallas import tpu as pltpu
from jax.experimental.pallas import tpu_sc as plsc

N, E, D = 10000, 320000, 128
NC, NS = 2, 16
NW = NC * NS
NPAD = NS * 640
ROWS_T = 624
ROWS_TAIL = N - NS * ROWS_T
K = 128
NCHUNK = E // K
GFULL = NCHUNK // NW
GREM = NCHUNK - GFULL * NW
EPS = 1e-5

_mesh = plsc.VectorSubcoreMesh(
    core_axis_name="c", subcore_axis_name="s", num_cores=NC, num_subcores=NS
)


@functools.partial(
    pl.kernel,
    out_type=jax.ShapeDtypeStruct((NC * NPAD,), jnp.float32),
    mesh=_mesh,
    scratch_types=[
        pltpu.VMEM_SHARED((NPAD,), jnp.float32),
        pltpu.VMEM((640,), jnp.float32),
        pltpu.VMEM((K,), jnp.int32),
        pltpu.VMEM((K,), jnp.float32),
    ],
)
def _deg_kernel(col_hbm, out_hbm, deg_sh, stage_v, idx_v, ones_v):
    c = lax.axis_index("c")
    s = lax.axis_index("s")
    t = c * NS + s
    ones16 = jnp.ones((16,), jnp.float32)
    for j in range(K // 16):
        ones_v[pl.ds(j * 16, 16)] = ones16
    initv = jnp.where(c == 0, 1.0, 0.0).astype(jnp.float32) * ones16
    for j in range(640 // 16):
        stage_v[pl.ds(j * 16, 16)] = initv
    pltpu.sync_copy(stage_v, deg_sh.at[pl.ds(s * 640, 640)])
    plsc.subcore_barrier()

    def chunk(j, carry):
        off = (t + j * NW) * K
        pltpu.sync_copy(col_hbm.at[pl.ds(off, K)], idx_v)
        pltpu.sync_copy(ones_v, deg_sh.at[idx_v], add=True)
        return carry

    lax.fori_loop(0, GFULL, chunk, 0)

    @pl.when(t < GREM)
    def _():
        chunk(GFULL, 0)

    plsc.subcore_barrier()
    pltpu.sync_copy(deg_sh.at[pl.ds(s * 640, 640)], stage_v)
    pltpu.sync_copy(stage_v, out_hbm.at[pl.ds(c * NPAD + s * 640, 640)])


@functools.partial(
    pl.kernel,
    out_type=jax.ShapeDtypeStruct((NC * N, D), jnp.float32),
    mesh=_mesh,
    scratch_types=[
        pltpu.VMEM_SHARED((N, D), jnp.float32),
        pltpu.VMEM((ROWS_TAIL, D), jnp.float32),
        pltpu.VMEM((K,), jnp.int32),
        pltpu.VMEM((K,), jnp.int32),
        pltpu.VMEM((K, D), jnp.float32),
        pltpu.SemaphoreType.DMA,
    ],
)
def _msg_kernel(y_hbm, row_hbm, col_hbm, out_hbm, acc_sh, stage_x,
                ridx_v, cidx_v, gath_v, sem):
    c = lax.axis_index("c")
    s = lax.axis_index("s")
    t = c * NS + s
    NROW_CHUNKS = N // K

    def stage_init(j, carry):
        cid = s + j * NS
        @pl.when(cid < NROW_CHUNKS)
        def _():
            row0 = cid * K
            pltpu.sync_copy(y_hbm.at[pl.ds(row0, K)], gath_v)
            pltpu.sync_copy(gath_v, acc_sh.at[pl.ds(row0, K)])
        return carry

    lax.fori_loop(0, (NROW_CHUNKS + NS - 1) // NS, stage_init, 0)

    @pl.when(s == NS - 1)
    def _():
        pltpu.sync_copy(y_hbm.at[pl.ds(NROW_CHUNKS * K, ROWS_TAIL)], stage_x)
        pltpu.sync_copy(stage_x, acc_sh.at[pl.ds(NROW_CHUNKS * K, ROWS_TAIL)])

    plsc.subcore_barrier()

    def chunk(j, carry):
        off = (t + j * NW) * K
        pltpu.sync_copy(row_hbm.at[pl.ds(off, K)], ridx_v)
        pltpu.sync_copy(col_hbm.at[pl.ds(off, K)], cidx_v)
        pltpu.async_copy(y_hbm.at[ridx_v], gath_v, sem).wait()
        pltpu.sync_copy(gath_v, acc_sh.at[cidx_v], add=True)
        return carry

    lax.fori_loop(0, GFULL, chunk, 0)

    @pl.when(t < GREM)
    def _():
        chunk(GFULL, 0)

    plsc.subcore_barrier()

    def stage_out(j, carry):
        cid = s + j * NS
        @pl.when(cid < NROW_CHUNKS)
        def _():
            row0 = cid * K
            pltpu.sync_copy(acc_sh.at[pl.ds(row0, K)], gath_v)
            pltpu.sync_copy(gath_v, out_hbm.at[pl.ds(c * N + row0, K)])
        return carry

    lax.fori_loop(0, (NROW_CHUNKS + NS - 1) // NS, stage_out, 0)

    @pl.when(s == NS - 1)
    def _():
        pltpu.sync_copy(acc_sh.at[pl.ds(NROW_CHUNKS * K, ROWS_TAIL)], stage_x)
        pltpu.sync_copy(stage_x, out_hbm.at[pl.ds(c * N + NROW_CHUNKS * K, ROWS_TAIL)])


def _dis_from_parts(degp):
    deg = degp.reshape(NC, NPAD)[:, :N].sum(axis=0)
    return lax.rsqrt(deg)[:, None]


def _tc1_body(x_ref, w_ref, degp_ref, y_ref):
    dis = _dis_from_parts(degp_ref[...])
    xw = jnp.dot(x_ref[...], w_ref[...], preferred_element_type=jnp.float32)
    y_ref[...] = dis * xw


_tc1 = pl.pallas_call(
    _tc1_body, out_shape=jax.ShapeDtypeStruct((N, D), jnp.float32)
)


def _norm_act(acc_cat, y, dis, b, g, be, a):
    h = dis * (acc_cat[:N] + acc_cat[N:] - y) + b
    mean = jnp.mean(h, axis=0)
    var = jnp.mean((h - mean) ** 2, axis=0)
    hn = g * (h - mean) / jnp.sqrt(var + EPS) + be
    return jnp.where(hn >= 0, hn, a * hn)


def _tc_mid_body(acc_ref, y_ref, degp_ref, b_ref, g_ref, be_ref, a_ref, w_ref,
                 o_ref):
    dis = _dis_from_parts(degp_ref[...])
    hp = _norm_act(acc_ref[...], y_ref[...], dis, b_ref[...], g_ref[...],
                   be_ref[...], a_ref[...])
    xw = jnp.dot(hp, w_ref[...], preferred_element_type=jnp.float32)
    o_ref[...] = dis * xw


_tc_mid = pl.pallas_call(
    _tc_mid_body, out_shape=jax.ShapeDtypeStruct((N, D), jnp.float32)
)


def _tc_out_body(acc_ref, y_ref, degp_ref, b_ref, g_ref, be_ref, a_ref, o_ref):
    dis = _dis_from_parts(degp_ref[...])
    o_ref[...] = _norm_act(acc_ref[...], y_ref[...], dis, b_ref[...], g_ref[...],
                           be_ref[...], a_ref[...])


_tc_out = pl.pallas_call(
    _tc_out_body, out_shape=jax.ShapeDtypeStruct((N, D), jnp.float32)
)


def kernel(x, edge_index, W1, b1, gamma1, beta1, a1, W2, b2, gamma2, beta2, a2):
    row = edge_index[0]
    col = edge_index[1]
    degp = _deg_kernel(col)
    b1r, g1r, be1r = b1.reshape(1, -1), gamma1.reshape(1, -1), beta1.reshape(1, -1)
    b2r, g2r, be2r = b2.reshape(1, -1), gamma2.reshape(1, -1), beta2.reshape(1, -1)
    a1r, a2r = a1.reshape(1, 1), a2.reshape(1, 1)

    y1 = _tc1(x, W1, degp)
    acc1 = _msg_kernel(y1, row, col)
    y2 = _tc_mid(acc1, y1, degp, b1r, g1r, be1r, a1r, W2)
    acc2 = _msg_kernel(y2, row, col)
    return _tc_out(acc2, y2, degp, b2r, g2r, be2r, a2r)


# speedup vs baseline: 24.9257x; 1.5165x over previous
import functools

import jax
import jax.numpy as jnp
from jax import lax
from jax.experimental import pallas as pl
from jax.experimental.pallas import tpu as pltpu
from jax.experimental.pallas import tpu_sc as plsc

N, E, D = 10000, 320000, 128
NC, NS = 2, 16
NW = NC * NS
NPAD = NS * 640
ROWS_T = 624
ROWS_TAIL = N - NS * ROWS_T
K = 128
NCHUNK = E // K
GFULL = NCHUNK // NW
GREM = NCHUNK - GFULL * NW
EPS = 1e-5

_mesh = plsc.VectorSubcoreMesh(
    core_axis_name="c", subcore_axis_name="s", num_cores=NC, num_subcores=NS
)


@functools.partial(
    pl.kernel,
    out_type=jax.ShapeDtypeStruct((NC * NPAD,), jnp.float32),
    mesh=_mesh,
    scratch_types=[
        pltpu.VMEM_SHARED((NPAD,), jnp.float32),
        pltpu.VMEM((640,), jnp.float32),
        pltpu.VMEM((K,), jnp.int32),
        pltpu.VMEM((K,), jnp.float32),
    ],
)
def _deg_kernel(col_hbm, out_hbm, deg_sh, stage_v, idx_v, ones_v):
    c = lax.axis_index("c")
    s = lax.axis_index("s")
    t = c * NS + s
    ones16 = jnp.ones((16,), jnp.float32)
    for j in range(K // 16):
        ones_v[pl.ds(j * 16, 16)] = ones16
    initv = jnp.where(c == 0, 1.0, 0.0).astype(jnp.float32) * ones16
    for j in range(640 // 16):
        stage_v[pl.ds(j * 16, 16)] = initv
    pltpu.sync_copy(stage_v, deg_sh.at[pl.ds(s * 640, 640)])
    plsc.subcore_barrier()

    def chunk(j, carry):
        off = (t + j * NW) * K
        pltpu.sync_copy(col_hbm.at[pl.ds(off, K)], idx_v)
        pltpu.sync_copy(ones_v, deg_sh.at[idx_v], add=True)
        return carry

    lax.fori_loop(0, GFULL, chunk, 0)

    @pl.when(t < GREM)
    def _():
        chunk(GFULL, 0)

    plsc.subcore_barrier()
    pltpu.sync_copy(deg_sh.at[pl.ds(s * 640, 640)], stage_v)
    pltpu.sync_copy(stage_v, out_hbm.at[pl.ds(c * NPAD + s * 640, 640)])


@functools.partial(
    pl.kernel,
    out_type=jax.ShapeDtypeStruct((NC * N, D), jnp.float32),
    mesh=_mesh,
    scratch_types=[
        pltpu.VMEM_SHARED((N, D), jnp.float32),
        pltpu.VMEM((ROWS_TAIL, D), jnp.float32),
        pltpu.VMEM((K,), jnp.int32),
        pltpu.VMEM((K,), jnp.int32),
        pltpu.VMEM((K,), jnp.int32),
        pltpu.VMEM((K,), jnp.int32),
        pltpu.VMEM((K,), jnp.int32),
        pltpu.VMEM((K,), jnp.int32),
        pltpu.VMEM((K, D), jnp.float32),
        pltpu.VMEM((K, D), jnp.float32),
        pltpu.SemaphoreType.DMA,
        pltpu.SemaphoreType.DMA,
        pltpu.SemaphoreType.DMA,
        pltpu.SemaphoreType.DMA,
        pltpu.SemaphoreType.DMA,
        pltpu.SemaphoreType.DMA,
    ],
)
def _msg_kernel(y_hbm, row_hbm, col_hbm, out_hbm, acc_sh, stage_x,
                ridx0, ridx1, cidx0, cidx1, cidx0s, cidx1s, gath0, gath1,
                sem_i0, sem_i1, sem_g0, sem_g1, sem_s0, sem_s1):
    c = lax.axis_index("c")
    s = lax.axis_index("s")
    t = c * NS + s
    NROW_CHUNKS = N // K

    def stage_init(j, carry):
        cid = s + j * NS
        @pl.when(cid < NROW_CHUNKS)
        def _():
            row0 = cid * K
            pltpu.sync_copy(y_hbm.at[pl.ds(row0, K)], gath0)
            pltpu.sync_copy(gath0, acc_sh.at[pl.ds(row0, K)])
        return carry

    lax.fori_loop(0, (NROW_CHUNKS + NS - 1) // NS, stage_init, 0)

    @pl.when(s == NS - 1)
    def _():
        pltpu.sync_copy(y_hbm.at[pl.ds(NROW_CHUNKS * K, ROWS_TAIL)], stage_x)
        pltpu.sync_copy(stage_x, acc_sh.at[pl.ds(NROW_CHUNKS * K, ROWS_TAIL)])

    plsc.subcore_barrier()

    def idx_off(j):
        return (t + j * NW) * K

    def issue_idx(j, ridx, cidx, sem):
        off = idx_off(j)
        pltpu.async_copy(row_hbm.at[pl.ds(off, K)], ridx, sem)
        pltpu.async_copy(col_hbm.at[pl.ds(off, K)], cidx, sem)

    def wait_idx(ridx, cidx, sem):
        pltpu.make_async_copy(row_hbm.at[pl.ds(0, K)], ridx, sem).wait()
        pltpu.make_async_copy(col_hbm.at[pl.ds(0, K)], cidx, sem).wait()

    def vcopy(src, dst):
        for i in range(K // 16):
            sl = pl.ds(i * 16, 16)
            dst[sl] = src[sl]

    issue_idx(0, ridx0, cidx0, sem_i0)
    issue_idx(1, ridx1, cidx1, sem_i1)

    bufs = ((ridx0, cidx0, cidx0s, gath0, sem_i0, sem_g0, sem_s0),
            (ridx1, cidx1, cidx1s, gath1, sem_i1, sem_g1, sem_s1))

    def body(m, carry):
        for b, (ridx, cidx, cidxs, gath, sem_i, sem_g, sem_s) in enumerate(bufs):
            wait_idx(ridx, cidx, sem_i)

            @pl.when(m > 0)
            def _():
                pltpu.make_async_copy(gath, acc_sh.at[cidxs], sem_s).wait()

            pltpu.async_copy(y_hbm.at[ridx], gath, sem_g)
        for b, (ridx, cidx, cidxs, gath, sem_i, sem_g, sem_s) in enumerate(bufs):
            pltpu.make_async_copy(y_hbm.at[ridx], gath, sem_g).wait()
            vcopy(cidx, cidxs)
            pltpu.async_copy(gath, acc_sh.at[cidxs], sem_s, add=True)
            issue_idx(2 * m + b + 2, ridx, cidx, sem_i)
        return carry

    lax.fori_loop(0, GFULL // 2, body, 0)

    for ridx, cidx, cidxs, gath, sem_i, sem_g, sem_s in bufs:
        wait_idx(ridx, cidx, sem_i)
        pltpu.make_async_copy(gath, acc_sh.at[cidxs], sem_s).wait()

    @pl.when(t < GREM)
    def _():
        pltpu.async_copy(y_hbm.at[ridx0], gath0, sem_g0).wait()
        pltpu.sync_copy(gath0, acc_sh.at[cidx0], add=True)

    plsc.subcore_barrier()

    def stage_out(j, carry):
        cid = s + j * NS
        @pl.when(cid < NROW_CHUNKS)
        def _():
            row0 = cid * K
            pltpu.sync_copy(acc_sh.at[pl.ds(row0, K)], gath0)
            pltpu.sync_copy(gath0, out_hbm.at[pl.ds(c * N + row0, K)])
        return carry

    lax.fori_loop(0, (NROW_CHUNKS + NS - 1) // NS, stage_out, 0)

    @pl.when(s == NS - 1)
    def _():
        pltpu.sync_copy(acc_sh.at[pl.ds(NROW_CHUNKS * K, ROWS_TAIL)], stage_x)
        pltpu.sync_copy(stage_x, out_hbm.at[pl.ds(c * N + NROW_CHUNKS * K, ROWS_TAIL)])


def _dis_from_parts(degp):
    deg = degp.reshape(NC, NPAD)[:, :N].sum(axis=0)
    return lax.rsqrt(deg)[:, None]


def _tc1_body(x_ref, w_ref, degp_ref, y_ref):
    dis = _dis_from_parts(degp_ref[...])
    xw = jnp.dot(x_ref[...], w_ref[...], preferred_element_type=jnp.float32)
    y_ref[...] = dis * xw


_tc1 = pl.pallas_call(
    _tc1_body, out_shape=jax.ShapeDtypeStruct((N, D), jnp.float32)
)


def _norm_act(acc_cat, y, dis, b, g, be, a):
    h = dis * (acc_cat[:N] + acc_cat[N:] - y) + b
    mean = jnp.mean(h, axis=0)
    var = jnp.mean((h - mean) ** 2, axis=0)
    hn = g * (h - mean) / jnp.sqrt(var + EPS) + be
    return jnp.where(hn >= 0, hn, a * hn)


def _tc_mid_body(acc_ref, y_ref, degp_ref, b_ref, g_ref, be_ref, a_ref, w_ref,
                 o_ref):
    dis = _dis_from_parts(degp_ref[...])
    hp = _norm_act(acc_ref[...], y_ref[...], dis, b_ref[...], g_ref[...],
                   be_ref[...], a_ref[...])
    xw = jnp.dot(hp, w_ref[...], preferred_element_type=jnp.float32)
    o_ref[...] = dis * xw


_tc_mid = pl.pallas_call(
    _tc_mid_body, out_shape=jax.ShapeDtypeStruct((N, D), jnp.float32)
)


def _tc_out_body(acc_ref, y_ref, degp_ref, b_ref, g_ref, be_ref, a_ref, o_ref):
    dis = _dis_from_parts(degp_ref[...])
    o_ref[...] = _norm_act(acc_ref[...], y_ref[...], dis, b_ref[...], g_ref[...],
                           be_ref[...], a_ref[...])


_tc_out = pl.pallas_call(
    _tc_out_body, out_shape=jax.ShapeDtypeStruct((N, D), jnp.float32)
)


def kernel(x, edge_index, W1, b1, gamma1, beta1, a1, W2, b2, gamma2, beta2, a2):
    pad = jnp.zeros(((GFULL + 2) * NW - NCHUNK) * K, dtype=jnp.int32)
    row = jnp.concatenate([edge_index[0], pad])
    col = jnp.concatenate([edge_index[1], pad])
    degp = _deg_kernel(col)
    b1r, g1r, be1r = b1.reshape(1, -1), gamma1.reshape(1, -1), beta1.reshape(1, -1)
    b2r, g2r, be2r = b2.reshape(1, -1), gamma2.reshape(1, -1), beta2.reshape(1, -1)
    a1r, a2r = a1.reshape(1, 1), a2.reshape(1, 1)

    y1 = _tc1(x, W1, degp)
    acc1 = _msg_kernel(y1, row, col)
    y2 = _tc_mid(acc1, y1, degp, b1r, g1r, be1r, a1r, W2)
    acc2 = _msg_kernel(y2, row, col)
    return _tc_out(acc2, y2, degp, b2r, g2r, be2r, a2r)


# speedup vs baseline: 28.0221x; 1.1242x over previous
import functools

import jax
import jax.numpy as jnp
from jax import lax
from jax.experimental import pallas as pl
from jax.experimental.pallas import tpu as pltpu
from jax.experimental.pallas import tpu_sc as plsc

N, E, D = 10000, 320000, 128
NC, NS = 2, 16
NW = NC * NS
NPAD = NS * 640
ROWS_T = 624
ROWS_TAIL = N - NS * ROWS_T
K = 128
NCHUNK = E // K
GFULL = NCHUNK // NW
GREM = NCHUNK - GFULL * NW
NBUF = 3
EPS = 1e-5

_mesh = plsc.VectorSubcoreMesh(
    core_axis_name="c", subcore_axis_name="s", num_cores=NC, num_subcores=NS
)


@functools.partial(
    pl.kernel,
    out_type=jax.ShapeDtypeStruct((NC * NPAD,), jnp.float32),
    mesh=_mesh,
    scratch_types=[
        pltpu.VMEM_SHARED((NPAD,), jnp.float32),
        pltpu.VMEM((640,), jnp.float32),
        pltpu.VMEM((K,), jnp.int32),
        pltpu.VMEM((K,), jnp.float32),
    ],
)
def _deg_kernel(col_hbm, out_hbm, deg_sh, stage_v, idx_v, ones_v):
    c = lax.axis_index("c")
    s = lax.axis_index("s")
    t = c * NS + s
    ones16 = jnp.ones((16,), jnp.float32)
    for j in range(K // 16):
        ones_v[pl.ds(j * 16, 16)] = ones16
    initv = jnp.where(c == 0, 1.0, 0.0).astype(jnp.float32) * ones16
    for j in range(640 // 16):
        stage_v[pl.ds(j * 16, 16)] = initv
    pltpu.sync_copy(stage_v, deg_sh.at[pl.ds(s * 640, 640)])
    plsc.subcore_barrier()

    def chunk(j, carry):
        off = (t + j * NW) * K
        pltpu.sync_copy(col_hbm.at[pl.ds(off, K)], idx_v)
        pltpu.sync_copy(ones_v, deg_sh.at[idx_v], add=True)
        return carry

    lax.fori_loop(0, GFULL, chunk, 0)

    @pl.when(t < GREM)
    def _():
        chunk(GFULL, 0)

    plsc.subcore_barrier()
    pltpu.sync_copy(deg_sh.at[pl.ds(s * 640, 640)], stage_v)
    pltpu.sync_copy(stage_v, out_hbm.at[pl.ds(c * NPAD + s * 640, 640)])


@functools.partial(
    pl.kernel,
    out_type=jax.ShapeDtypeStruct((NC * N, D), jnp.float32),
    mesh=_mesh,
    scratch_types=[
        pltpu.VMEM_SHARED((N, D), jnp.float32),
        [pltpu.VMEM((K,), jnp.int32) for _ in range(NBUF)],
        [pltpu.VMEM((K,), jnp.int32) for _ in range(NBUF)],
        [pltpu.VMEM((K,), jnp.int32) for _ in range(NBUF)],
        [pltpu.VMEM((K, D), jnp.float32) for _ in range(NBUF)],
        [pltpu.SemaphoreType.DMA for _ in range(NBUF)],
        [pltpu.SemaphoreType.DMA for _ in range(NBUF)],
        [pltpu.SemaphoreType.DMA for _ in range(NBUF)],
    ],
)
def _msg_kernel(y_hbm, row_hbm, col_hbm, out_hbm, acc_sh,
                ridxs, cidxs, cidxss, gaths, sem_is, sem_gs, sem_ss):
    c = lax.axis_index("c")
    s = lax.axis_index("s")
    t = c * NS + s
    gath0 = gaths[0]
    NROW_CHUNKS = N // K

    def stage_init(j, carry):
        cid = s + j * NS
        @pl.when(cid < NROW_CHUNKS)
        def _():
            row0 = cid * K
            pltpu.sync_copy(y_hbm.at[pl.ds(row0, K)], gath0)
            pltpu.sync_copy(gath0, acc_sh.at[pl.ds(row0, K)])
        return carry

    lax.fori_loop(0, (NROW_CHUNKS + NS - 1) // NS, stage_init, 0)

    @pl.when(s == NS - 1)
    def _():
        tail0 = NROW_CHUNKS * K
        pltpu.sync_copy(y_hbm.at[pl.ds(tail0, ROWS_TAIL)],
                        gath0.at[pl.ds(0, ROWS_TAIL)])
        pltpu.sync_copy(gath0.at[pl.ds(0, ROWS_TAIL)],
                        acc_sh.at[pl.ds(tail0, ROWS_TAIL)])

    plsc.subcore_barrier()

    def issue_idx(j, ridx, cidx, sem):
        off = (t + j * NW) * K
        pltpu.async_copy(row_hbm.at[pl.ds(off, K)], ridx, sem)
        pltpu.async_copy(col_hbm.at[pl.ds(off, K)], cidx, sem)

    def wait_idx(ridx, cidx, sem):
        pltpu.make_async_copy(row_hbm.at[pl.ds(0, K)], ridx, sem).wait()
        pltpu.make_async_copy(col_hbm.at[pl.ds(0, K)], cidx, sem).wait()

    def vcopy(src, dst):
        for i in range(K // 16):
            sl = pl.ds(i * 16, 16)
            dst[sl] = src[sl]

    bufs = tuple(zip(ridxs, cidxs, cidxss, gaths, sem_is, sem_gs, sem_ss))
    for b, (ridx, cidx, cidxs_, gath, sem_i, sem_g, sem_s) in enumerate(bufs):
        issue_idx(b, ridx, cidx, sem_i)

    def body(m, carry):
        for b, (ridx, cidx, cidxs_, gath, sem_i, sem_g, sem_s) in enumerate(bufs):
            wait_idx(ridx, cidx, sem_i)

            @pl.when(m > 0)
            def _():
                pltpu.make_async_copy(gath, acc_sh.at[cidxs_], sem_s).wait()

            pltpu.async_copy(y_hbm.at[ridx], gath, sem_g)
        for b, (ridx, cidx, cidxs_, gath, sem_i, sem_g, sem_s) in enumerate(bufs):
            pltpu.make_async_copy(y_hbm.at[ridx], gath, sem_g).wait()
            vcopy(cidx, cidxs_)
            pltpu.async_copy(gath, acc_sh.at[cidxs_], sem_s, add=True)
            issue_idx(NBUF * m + b + NBUF, ridx, cidx, sem_i)
        return carry

    lax.fori_loop(0, GFULL // NBUF, body, 0)

    for ridx, cidx, cidxs_, gath, sem_i, sem_g, sem_s in bufs:
        wait_idx(ridx, cidx, sem_i)
        pltpu.make_async_copy(gath, acc_sh.at[cidxs_], sem_s).wait()

    @pl.when(t < GREM)
    def _():
        pltpu.async_copy(y_hbm.at[ridxs[0]], gaths[0], sem_gs[0]).wait()
        pltpu.sync_copy(gaths[0], acc_sh.at[cidxs[0]], add=True)

    plsc.subcore_barrier()

    def stage_out(j, carry):
        cid = s + j * NS
        @pl.when(cid < NROW_CHUNKS)
        def _():
            row0 = cid * K
            pltpu.sync_copy(acc_sh.at[pl.ds(row0, K)], gath0)
            pltpu.sync_copy(gath0, out_hbm.at[pl.ds(c * N + row0, K)])
        return carry

    lax.fori_loop(0, (NROW_CHUNKS + NS - 1) // NS, stage_out, 0)

    @pl.when(s == NS - 1)
    def _():
        tail0 = NROW_CHUNKS * K
        pltpu.sync_copy(acc_sh.at[pl.ds(tail0, ROWS_TAIL)],
                        gath0.at[pl.ds(0, ROWS_TAIL)])
        pltpu.sync_copy(gath0.at[pl.ds(0, ROWS_TAIL)],
                        out_hbm.at[pl.ds(c * N + tail0, ROWS_TAIL)])


def _dis_from_parts(degp):
    deg = degp.reshape(NC, NPAD)[:, :N].sum(axis=0)
    return lax.rsqrt(deg)[:, None]


def _tc1_body(x_ref, w_ref, degp_ref, y_ref):
    dis = _dis_from_parts(degp_ref[...])
    xw = jnp.dot(x_ref[...], w_ref[...], preferred_element_type=jnp.float32)
    y_ref[...] = dis * xw


_tc1 = pl.pallas_call(
    _tc1_body, out_shape=jax.ShapeDtypeStruct((N, D), jnp.float32)
)


def _norm_act(acc_cat, y, dis, b, g, be, a):
    h = dis * (acc_cat[:N] + acc_cat[N:] - y) + b
    mean = jnp.mean(h, axis=0)
    var = jnp.mean((h - mean) ** 2, axis=0)
    hn = g * (h - mean) / jnp.sqrt(var + EPS) + be
    return jnp.where(hn >= 0, hn, a * hn)


def _tc_mid_body(acc_ref, y_ref, degp_ref, b_ref, g_ref, be_ref, a_ref, w_ref,
                 o_ref):
    dis = _dis_from_parts(degp_ref[...])
    hp = _norm_act(acc_ref[...], y_ref[...], dis, b_ref[...], g_ref[...],
                   be_ref[...], a_ref[...])
    xw = jnp.dot(hp, w_ref[...], preferred_element_type=jnp.float32)
    o_ref[...] = dis * xw


_tc_mid = pl.pallas_call(
    _tc_mid_body, out_shape=jax.ShapeDtypeStruct((N, D), jnp.float32)
)


def _tc_out_body(acc_ref, y_ref, degp_ref, b_ref, g_ref, be_ref, a_ref, o_ref):
    dis = _dis_from_parts(degp_ref[...])
    o_ref[...] = _norm_act(acc_ref[...], y_ref[...], dis, b_ref[...], g_ref[...],
                           be_ref[...], a_ref[...])


_tc_out = pl.pallas_call(
    _tc_out_body, out_shape=jax.ShapeDtypeStruct((N, D), jnp.float32)
)


def kernel(x, edge_index, W1, b1, gamma1, beta1, a1, W2, b2, gamma2, beta2, a2):
    pad = jnp.zeros(((GFULL + NBUF) * NW - NCHUNK) * K, dtype=jnp.int32)
    row = jnp.concatenate([edge_index[0], pad])
    col = jnp.concatenate([edge_index[1], pad])
    degp = _deg_kernel(col)
    b1r, g1r, be1r = b1.reshape(1, -1), gamma1.reshape(1, -1), beta1.reshape(1, -1)
    b2r, g2r, be2r = b2.reshape(1, -1), gamma2.reshape(1, -1), beta2.reshape(1, -1)
    a1r, a2r = a1.reshape(1, 1), a2.reshape(1, 1)

    y1 = _tc1(x, W1, degp)
    acc1 = _msg_kernel(y1, row, col)
    y2 = _tc_mid(acc1, y1, degp, b1r, g1r, be1r, a1r, W2)
    acc2 = _msg_kernel(y2, row, col)
    return _tc_out(acc2, y2, degp, b2r, g2r, be2r, a2r)


# speedup vs baseline: 30.1578x; 1.0762x over previous
import functools

import jax
import jax.numpy as jnp
from jax import lax
from jax.experimental import pallas as pl
from jax.experimental.pallas import tpu as pltpu
from jax.experimental.pallas import tpu_sc as plsc

N, E, D = 10000, 320000, 128
NC, NS = 2, 16
NW = NC * NS
NPAD = NS * 640
ROWS_T = 624
ROWS_TAIL = N - NS * ROWS_T
K = 128
NCHUNK = E // K
GFULL = NCHUNK // NW
GREM = NCHUNK - GFULL * NW
NBUF = 3
EPS = 1e-5

_mesh = plsc.VectorSubcoreMesh(
    core_axis_name="c", subcore_axis_name="s", num_cores=NC, num_subcores=NS
)


@functools.partial(
    pl.kernel,
    out_type=jax.ShapeDtypeStruct((NC * NPAD,), jnp.float32),
    mesh=_mesh,
    scratch_types=[
        pltpu.VMEM_SHARED((NPAD,), jnp.float32),
        pltpu.VMEM((640,), jnp.float32),
        pltpu.VMEM((K,), jnp.float32),
        [pltpu.VMEM((K,), jnp.int32) for _ in range(2)],
        [pltpu.VMEM((K,), jnp.int32) for _ in range(2)],
        [pltpu.SemaphoreType.DMA for _ in range(2)],
        [pltpu.SemaphoreType.DMA for _ in range(2)],
    ],
)
def _deg_kernel(col_hbm, out_hbm, deg_sh, stage_v, ones_v, cidxs, cidxss,
                sem_is, sem_ss):
    c = lax.axis_index("c")
    s = lax.axis_index("s")
    t = c * NS + s
    ones16 = jnp.ones((16,), jnp.float32)
    for j in range(K // 16):
        ones_v[pl.ds(j * 16, 16)] = ones16
    initv = jnp.where(c == 0, 1.0, 0.0).astype(jnp.float32) * ones16
    for j in range(640 // 16):
        stage_v[pl.ds(j * 16, 16)] = initv
    pltpu.sync_copy(stage_v, deg_sh.at[pl.ds(s * 640, 640)])
    plsc.subcore_barrier()

    def issue_idx(j, cidx, sem):
        off = (t + j * NW) * K
        pltpu.async_copy(col_hbm.at[pl.ds(off, K)], cidx, sem)

    def vcopy(src, dst):
        for i in range(K // 16):
            sl = pl.ds(i * 16, 16)
            dst[sl] = src[sl]

    bufs = tuple(zip(cidxs, cidxss, sem_is, sem_ss))
    for b, (cidx, cidxs_, sem_i, sem_s) in enumerate(bufs):
        issue_idx(b, cidx, sem_i)

    def body(m, carry):
        for b, (cidx, cidxs_, sem_i, sem_s) in enumerate(bufs):
            pltpu.make_async_copy(col_hbm.at[pl.ds(0, K)], cidx, sem_i).wait()

            @pl.when(m > 0)
            def _():
                pltpu.make_async_copy(ones_v, deg_sh.at[cidxs_], sem_s).wait()

            vcopy(cidx, cidxs_)
            pltpu.async_copy(ones_v, deg_sh.at[cidxs_], sem_s, add=True)
            issue_idx(2 * m + b + 2, cidx, sem_i)
        return carry

    lax.fori_loop(0, GFULL // 2, body, 0)

    for cidx, cidxs_, sem_i, sem_s in bufs:
        pltpu.make_async_copy(col_hbm.at[pl.ds(0, K)], cidx, sem_i).wait()
        pltpu.make_async_copy(ones_v, deg_sh.at[cidxs_], sem_s).wait()

    @pl.when(t < GREM)
    def _():
        pltpu.sync_copy(ones_v, deg_sh.at[cidxs[0]], add=True)

    plsc.subcore_barrier()
    pltpu.sync_copy(deg_sh.at[pl.ds(s * 640, 640)], stage_v)
    pltpu.sync_copy(stage_v, out_hbm.at[pl.ds(c * NPAD + s * 640, 640)])


@functools.partial(
    pl.kernel,
    out_type=jax.ShapeDtypeStruct((NC * N, D), jnp.float32),
    mesh=_mesh,
    scratch_types=[
        pltpu.VMEM_SHARED((N, D), jnp.float32),
        [pltpu.VMEM((K,), jnp.int32) for _ in range(NBUF)],
        [pltpu.VMEM((K,), jnp.int32) for _ in range(NBUF)],
        [pltpu.VMEM((K,), jnp.int32) for _ in range(NBUF)],
        [pltpu.VMEM((K, D), jnp.float32) for _ in range(NBUF)],
        [pltpu.SemaphoreType.DMA for _ in range(NBUF)],
        [pltpu.SemaphoreType.DMA for _ in range(NBUF)],
        [pltpu.SemaphoreType.DMA for _ in range(NBUF)],
    ],
)
def _msg_kernel(y_hbm, row_hbm, col_hbm, out_hbm, acc_sh,
                ridxs, cidxs, cidxss, gaths, sem_is, sem_gs, sem_ss):
    c = lax.axis_index("c")
    s = lax.axis_index("s")
    t = c * NS + s
    gath0 = gaths[0]
    NROW_CHUNKS = N // K

    def stage_init(j, carry):
        cid = s + j * NS
        @pl.when(cid < NROW_CHUNKS)
        def _():
            row0 = cid * K
            pltpu.sync_copy(y_hbm.at[pl.ds(row0, K)], gath0)
            pltpu.sync_copy(gath0, acc_sh.at[pl.ds(row0, K)])
        return carry

    lax.fori_loop(0, (NROW_CHUNKS + NS - 1) // NS, stage_init, 0)

    @pl.when(s == NS - 1)
    def _():
        tail0 = NROW_CHUNKS * K
        pltpu.sync_copy(y_hbm.at[pl.ds(tail0, ROWS_TAIL)],
                        gath0.at[pl.ds(0, ROWS_TAIL)])
        pltpu.sync_copy(gath0.at[pl.ds(0, ROWS_TAIL)],
                        acc_sh.at[pl.ds(tail0, ROWS_TAIL)])

    plsc.subcore_barrier()

    def issue_idx(j, ridx, cidx, sem):
        off = (t + j * NW) * K
        pltpu.async_copy(row_hbm.at[pl.ds(off, K)], ridx, sem)
        pltpu.async_copy(col_hbm.at[pl.ds(off, K)], cidx, sem)

    def wait_idx(ridx, cidx, sem):
        pltpu.make_async_copy(row_hbm.at[pl.ds(0, K)], ridx, sem).wait()
        pltpu.make_async_copy(col_hbm.at[pl.ds(0, K)], cidx, sem).wait()

    def vcopy(src, dst):
        for i in range(K // 16):
            sl = pl.ds(i * 16, 16)
            dst[sl] = src[sl]

    bufs = tuple(zip(ridxs, cidxs, cidxss, gaths, sem_is, sem_gs, sem_ss))
    for b, (ridx, cidx, cidxs_, gath, sem_i, sem_g, sem_s) in enumerate(bufs):
        issue_idx(b, ridx, cidx, sem_i)

    def body(m, carry):
        for b, (ridx, cidx, cidxs_, gath, sem_i, sem_g, sem_s) in enumerate(bufs):
            wait_idx(ridx, cidx, sem_i)

            @pl.when(m > 0)
            def _():
                pltpu.make_async_copy(gath, acc_sh.at[cidxs_], sem_s).wait()

            pltpu.async_copy(y_hbm.at[ridx], gath, sem_g)
        for b, (ridx, cidx, cidxs_, gath, sem_i, sem_g, sem_s) in enumerate(bufs):
            pltpu.make_async_copy(y_hbm.at[ridx], gath, sem_g).wait()
            vcopy(cidx, cidxs_)
            pltpu.async_copy(gath, acc_sh.at[cidxs_], sem_s, add=True)
            issue_idx(NBUF * m + b + NBUF, ridx, cidx, sem_i)
        return carry

    lax.fori_loop(0, GFULL // NBUF, body, 0)

    for ridx, cidx, cidxs_, gath, sem_i, sem_g, sem_s in bufs:
        wait_idx(ridx, cidx, sem_i)
        pltpu.make_async_copy(gath, acc_sh.at[cidxs_], sem_s).wait()

    @pl.when(t < GREM)
    def _():
        pltpu.async_copy(y_hbm.at[ridxs[0]], gaths[0], sem_gs[0]).wait()
        pltpu.sync_copy(gaths[0], acc_sh.at[cidxs[0]], add=True)

    plsc.subcore_barrier()

    def stage_out(j, carry):
        cid = s + j * NS
        @pl.when(cid < NROW_CHUNKS)
        def _():
            row0 = cid * K
            pltpu.sync_copy(acc_sh.at[pl.ds(row0, K)], gath0)
            pltpu.sync_copy(gath0, out_hbm.at[pl.ds(c * N + row0, K)])
        return carry

    lax.fori_loop(0, (NROW_CHUNKS + NS - 1) // NS, stage_out, 0)

    @pl.when(s == NS - 1)
    def _():
        tail0 = NROW_CHUNKS * K
        pltpu.sync_copy(acc_sh.at[pl.ds(tail0, ROWS_TAIL)],
                        gath0.at[pl.ds(0, ROWS_TAIL)])
        pltpu.sync_copy(gath0.at[pl.ds(0, ROWS_TAIL)],
                        out_hbm.at[pl.ds(c * N + tail0, ROWS_TAIL)])


def _dis_from_parts(degp):
    deg = degp.reshape(NC, NPAD)[:, :N].sum(axis=0)
    return lax.rsqrt(deg)[:, None]


def _tc1_body(x_ref, w_ref, degp_ref, y_ref):
    dis = _dis_from_parts(degp_ref[...])
    xw = jnp.dot(x_ref[...], w_ref[...], preferred_element_type=jnp.float32)
    y_ref[...] = dis * xw


_tc1 = pl.pallas_call(
    _tc1_body, out_shape=jax.ShapeDtypeStruct((N, D), jnp.float32)
)


def _norm_act(acc_cat, y, dis, b, g, be, a):
    h = dis * (acc_cat[:N] + acc_cat[N:] - y) + b
    mean = jnp.mean(h, axis=0)
    var = jnp.mean((h - mean) ** 2, axis=0)
    hn = g * (h - mean) / jnp.sqrt(var + EPS) + be
    return jnp.where(hn >= 0, hn, a * hn)


def _tc_mid_body(acc_ref, y_ref, degp_ref, b_ref, g_ref, be_ref, a_ref, w_ref,
                 o_ref):
    dis = _dis_from_parts(degp_ref[...])
    hp = _norm_act(acc_ref[...], y_ref[...], dis, b_ref[...], g_ref[...],
                   be_ref[...], a_ref[...])
    xw = jnp.dot(hp, w_ref[...], preferred_element_type=jnp.float32)
    o_ref[...] = dis * xw


_tc_mid = pl.pallas_call(
    _tc_mid_body, out_shape=jax.ShapeDtypeStruct((N, D), jnp.float32)
)


def _tc_out_body(acc_ref, y_ref, degp_ref, b_ref, g_ref, be_ref, a_ref, o_ref):
    dis = _dis_from_parts(degp_ref[...])
    o_ref[...] = _norm_act(acc_ref[...], y_ref[...], dis, b_ref[...], g_ref[...],
                           be_ref[...], a_ref[...])


_tc_out = pl.pallas_call(
    _tc_out_body, out_shape=jax.ShapeDtypeStruct((N, D), jnp.float32)
)


def kernel(x, edge_index, W1, b1, gamma1, beta1, a1, W2, b2, gamma2, beta2, a2):
    pad = jnp.zeros(((GFULL + NBUF) * NW - NCHUNK) * K, dtype=jnp.int32)
    row = jnp.concatenate([edge_index[0], pad])
    col = jnp.concatenate([edge_index[1], pad])
    degp = _deg_kernel(col)
    b1r, g1r, be1r = b1.reshape(1, -1), gamma1.reshape(1, -1), beta1.reshape(1, -1)
    b2r, g2r, be2r = b2.reshape(1, -1), gamma2.reshape(1, -1), beta2.reshape(1, -1)
    a1r, a2r = a1.reshape(1, 1), a2.reshape(1, 1)

    y1 = _tc1(x, W1, degp)
    acc1 = _msg_kernel(y1, row, col)
    y2 = _tc_mid(acc1, y1, degp, b1r, g1r, be1r, a1r, W2)
    acc2 = _msg_kernel(y2, row, col)
    return _tc_out(acc2, y2, degp, b2r, g2r, be2r, a2r)
